# baseline (speedup 1.0000x reference)
import functools

import jax
import jax.numpy as jnp
from jax import lax
from jax.experimental import pallas as pl
from jax.experimental.pallas import tpu as pltpu
from jax.experimental.pallas import tpu_sc as plsc

NUM_HID = 64
BATCH = 1024
SEQ_LEN = 200

_NC = 2
_NS = 16
_NW = _NC * _NS
_SEQ_PER_W = BATCH // _NW
_CHUNK = 4
_NCHUNK = _SEQ_PER_W // _CHUNK
_ROWS = _CHUNK * SEQ_LEN

_SPB = 16
_PAIR = SEQ_LEN // 2


def _pos_encoding():
    positions = jnp.arange(SEQ_LEN, dtype=jnp.float32)[:, None]
    depth = NUM_HID / 2
    depths = jnp.arange(depth, dtype=jnp.float32)[None, :] / depth
    angle_rates = 1.0 / (10000.0 ** depths)
    angle_rads = positions * angle_rates
    return jnp.concatenate(
        [jnp.sin(angle_rads), jnp.cos(angle_rads)], axis=-1)


def _sc_body(x_hbm, tab_hbm, out_hbm, idx0, idx1, rows0, rows1,
             sem_g0, sem_g1, sem_s0, sem_s1):
    wid = lax.axis_index("s") * _NC + lax.axis_index("c")

    idxs = (idx0, idx1)
    rows = (rows0, rows1)
    sem_g = (sem_g0, sem_g1)
    sem_s = (sem_s0, sem_s1)
    gather_d = [None, None]
    scatter_d = [None, None]
    base_w = wid * _SEQ_PER_W * SEQ_LEN

    for g in range(_NCHUNK):
        b = g & 1
        base = base_w + g * _ROWS
        if scatter_d[b] is not None:
            scatter_d[b].wait()
        pltpu.sync_copy(x_hbm.at[pl.ds(base, _ROWS)], idxs[b])
        gather_d[b] = pltpu.async_copy(
            tab_hbm.at[idxs[b]], rows[b], sem_g[b])
        if g > 0:
            pb = 1 - b
            gather_d[pb].wait()
            pbase = base_w + (g - 1) * _ROWS
            scatter_d[pb] = pltpu.async_copy(
                rows[pb], out_hbm.at[pl.ds(pbase, _ROWS)], sem_s[pb])

    last = (_NCHUNK - 1) & 1
    gather_d[last].wait()
    lbase = base_w + (_NCHUNK - 1) * _ROWS
    scatter_d[last] = pltpu.async_copy(
        rows[last], out_hbm.at[pl.ds(lbase, _ROWS)], sem_s[last])
    scatter_d[1 - last].wait()
    scatter_d[last].wait()


def _tc_body(g_ref, pe_ref, o_ref):
    x = g_ref[...]
    y = x.reshape(_SPB, _PAIR, 128) + pe_ref[...][None]
    o_ref[:, pl.Slice(0, _PAIR, 2), :] = y[:, :, :NUM_HID]
    o_ref[:, pl.Slice(1, _PAIR, 2), :] = y[:, :, NUM_HID:]


@jax.jit
def _run(x_flat, emb_table, pe_pair):
    mesh = plsc.VectorSubcoreMesh(
        core_axis_name="c", subcore_axis_name="s",
        num_cores=_NC, num_subcores=_NS)
    gathered = functools.partial(
        pl.kernel,
        out_type=jax.ShapeDtypeStruct((BATCH * SEQ_LEN, NUM_HID), jnp.float32),
        mesh=mesh,
        scratch_types=[
            pltpu.VMEM((_ROWS,), jnp.int32),
            pltpu.VMEM((_ROWS,), jnp.int32),
            pltpu.VMEM((_ROWS, NUM_HID), jnp.float32),
            pltpu.VMEM((_ROWS, NUM_HID), jnp.float32),
            pltpu.SemaphoreType.DMA,
            pltpu.SemaphoreType.DMA,
            pltpu.SemaphoreType.DMA,
            pltpu.SemaphoreType.DMA,
        ],
        compiler_params=pltpu.CompilerParams(use_tc_tiling_on_sc=False),
    )(_sc_body)(x_flat, emb_table)

    g2 = gathered.reshape(BATCH * SEQ_LEN * NUM_HID // 128, 128)
    return pl.pallas_call(
        _tc_body,
        grid=(BATCH // _SPB,),
        in_specs=[
            pl.BlockSpec((_SPB * _PAIR, 128), lambda i: (i, 0)),
            pl.BlockSpec((_PAIR, 128), lambda i: (0, 0)),
        ],
        out_specs=pl.BlockSpec((_SPB, SEQ_LEN, NUM_HID), lambda i: (i, 0, 0)),
        out_shape=jax.ShapeDtypeStruct((BATCH, SEQ_LEN, NUM_HID), jnp.float32),
    )(g2, pe_pair)


def kernel(x, emb_table):
    pe_pair = _pos_encoding().reshape(_PAIR, 128)
    x_flat = x.reshape(-1).astype(jnp.int32)
    return _run(x_flat, emb_table, pe_pair)
